# Initial kernel scaffold; baseline (speedup 1.0000x reference)
import jax
import jax.numpy as jnp
from jax.experimental import pallas as pl


def kernel(x2d, edge_index_2d, x1d, edge_index_1d, x_edge, params):
    raise NotImplementedError("write your pallas kernel here")


# speedup vs baseline: 1.5186x; 1.5186x over previous
import functools

import jax
import jax.numpy as jnp
from jax import lax
from jax.experimental import pallas as pl
from jax.experimental.pallas import tpu as pltpu
from jax.experimental.pallas import tpu_sc as plsc

_F32 = jnp.float32
_NTILE = 16
_NCORE = 2
_NW = _NTILE * _NCORE
_LN = 128


@functools.lru_cache(maxsize=None)
def _make_sc_agg(e_pad: int, n_acc: int, d: int):
    idx_rows = e_pad // _LN
    rows_per_tile = idx_rows // _NW
    n_chunks = rows_per_tile // 2
    acc_rows = n_acc // _NTILE
    nz = acc_rows // 64

    mesh = plsc.VectorSubcoreMesh(core_axis_name="c", subcore_axis_name="s")

    def body(g_hbm, src_hbm, dst_hbm, out_hbm, sidx, didx, rows, zbuf, acc, sem):
        c = lax.axis_index("c")
        s = lax.axis_index("s")
        zero16 = jnp.zeros((16,), _F32)
        for r in range(64):
            for j in range(8):
                zbuf[r, pl.ds(16 * j, 16)] = zero16

        def zstep(i, _):
            pltpu.sync_copy(zbuf, acc.at[pl.ds(s * acc_rows + i * 64, 64)])
            return 0

        lax.fori_loop(0, nz, zstep, 0)
        plsc.subcore_barrier()

        wid = s * _NCORE + c
        base = wid * rows_per_tile

        def estep(i, _):
            ro = base + i * 2
            pltpu.sync_copy(src_hbm.at[pl.ds(ro, 2)], sidx)
            pltpu.sync_copy(dst_hbm.at[pl.ds(ro, 2)], didx)
            g0 = pltpu.async_copy(g_hbm.at[sidx.at[0]], rows.at[pl.ds(0, _LN)], sem)
            g1 = pltpu.async_copy(g_hbm.at[sidx.at[1]], rows.at[pl.ds(_LN, _LN)], sem)
            g0.wait()
            g1.wait()
            pltpu.sync_copy(rows.at[pl.ds(0, _LN)], acc.at[didx.at[0]], add=True)
            pltpu.sync_copy(rows.at[pl.ds(_LN, _LN)], acc.at[didx.at[1]], add=True)
            return 0

        lax.fori_loop(0, n_chunks, estep, 0)
        plsc.subcore_barrier()

        def ostep(i, _):
            off = s * acc_rows + i * 64
            pltpu.sync_copy(acc.at[pl.ds(off, 64)],
                            out_hbm.at[pl.ds(c * n_acc + off, 64)])
            return 0

        lax.fori_loop(0, nz, ostep, 0)

    return pl.kernel(
        body,
        mesh=mesh,
        out_type=jax.ShapeDtypeStruct((2 * n_acc, d), _F32),
        scratch_types=[
            pltpu.VMEM((2, _LN), jnp.int32),
            pltpu.VMEM((2, _LN), jnp.int32),
            pltpu.VMEM((2 * _LN, d), _F32),
            pltpu.VMEM((64, d), _F32),
            pltpu.VMEM_SHARED((n_acc, d), _F32),
            pltpu.SemaphoreType.DMA,
        ],
    )


@functools.lru_cache(maxsize=None)
def _make_sc_cnt(e_pad: int, n_acc: int):
    idx_rows = e_pad // _LN
    rows_per_tile = idx_rows // _NW
    n_chunks = rows_per_tile // 2
    acc_rows = n_acc // _NTILE
    nz = acc_rows // 64

    mesh = plsc.VectorSubcoreMesh(core_axis_name="c", subcore_axis_name="s")

    def body(dst_hbm, out_hbm, didx, ones_v, zbuf, cnt, sem):
        c = lax.axis_index("c")
        s = lax.axis_index("s")
        zero16 = jnp.zeros((16,), _F32)
        frac16 = jnp.full((16,), 1.0 / 128.0, _F32)
        for r in range(64):
            for j in range(8):
                zbuf[r, pl.ds(16 * j, 16)] = zero16
        for r in range(_LN):
            for j in range(8):
                ones_v[r, pl.ds(16 * j, 16)] = frac16

        def zstep(i, _):
            pltpu.sync_copy(zbuf, cnt.at[pl.ds(s * acc_rows + i * 64, 64)])
            return 0

        lax.fori_loop(0, nz, zstep, 0)
        plsc.subcore_barrier()

        wid = s * _NCORE + c
        base = wid * rows_per_tile

        def estep(i, _):
            ro = base + i * 2
            pltpu.sync_copy(dst_hbm.at[pl.ds(ro, 2)], didx)
            pltpu.sync_copy(ones_v, cnt.at[didx.at[0]], add=True)
            pltpu.sync_copy(ones_v, cnt.at[didx.at[1]], add=True)
            return 0

        lax.fori_loop(0, n_chunks, estep, 0)
        plsc.subcore_barrier()

        def ostep(i, _):
            off = s * acc_rows + i * 64
            pltpu.sync_copy(cnt.at[pl.ds(off, 64)],
                            out_hbm.at[pl.ds(c * n_acc + off, 64)])
            return 0

        lax.fori_loop(0, nz, ostep, 0)

    return pl.kernel(
        body,
        mesh=mesh,
        out_type=jax.ShapeDtypeStruct((2 * n_acc, _LN), _F32),
        scratch_types=[
            pltpu.VMEM((2, _LN), jnp.int32),
            pltpu.VMEM((_LN, _LN), _F32),
            pltpu.VMEM((64, _LN), _F32),
            pltpu.VMEM_SHARED((n_acc, _LN), _F32),
            pltpu.SemaphoreType.DMA,
        ],
    )


@functools.lru_cache(maxsize=None)
def _make_sc_egather(e_pad: int, d: int):
    idx_rows = e_pad // _LN
    rows_per_tile = idx_rows // _NW
    n_chunks = rows_per_tile // 2

    mesh = plsc.VectorSubcoreMesh(core_axis_name="c", subcore_axis_name="s")

    def body(a_hbm, b_hbm, src_hbm, dst_hbm, oa_hbm, ob_hbm,
             sidx, didx, rows_a, rows_b, sem):
        c = lax.axis_index("c")
        s = lax.axis_index("s")
        wid = s * _NCORE + c
        base = wid * rows_per_tile

        def estep(i, _):
            ro = base + i * 2
            pltpu.sync_copy(src_hbm.at[pl.ds(ro, 2)], sidx)
            pltpu.sync_copy(dst_hbm.at[pl.ds(ro, 2)], didx)
            g0 = pltpu.async_copy(a_hbm.at[sidx.at[0]], rows_a.at[pl.ds(0, _LN)], sem)
            g1 = pltpu.async_copy(a_hbm.at[sidx.at[1]], rows_a.at[pl.ds(_LN, _LN)], sem)
            g2 = pltpu.async_copy(b_hbm.at[didx.at[0]], rows_b.at[pl.ds(0, _LN)], sem)
            g3 = pltpu.async_copy(b_hbm.at[didx.at[1]], rows_b.at[pl.ds(_LN, _LN)], sem)
            g0.wait()
            g1.wait()
            g2.wait()
            g3.wait()
            pltpu.sync_copy(rows_a, oa_hbm.at[pl.ds(ro * _LN, 2 * _LN)])
            pltpu.sync_copy(rows_b, ob_hbm.at[pl.ds(ro * _LN, 2 * _LN)])
            return 0

        lax.fori_loop(0, n_chunks, estep, 0)

    return pl.kernel(
        body,
        mesh=mesh,
        out_type=(jax.ShapeDtypeStruct((e_pad, d), _F32),
                  jax.ShapeDtypeStruct((e_pad, d), _F32)),
        scratch_types=[
            pltpu.VMEM((2, _LN), jnp.int32),
            pltpu.VMEM((2, _LN), jnp.int32),
            pltpu.VMEM((2 * _LN, d), _F32),
            pltpu.VMEM((2 * _LN, d), _F32),
            pltpu.SemaphoreType.DMA,
        ],
    )


def _pad_edges(src, dst, e_pad, dummy_dst):
    e = src.shape[0]
    pad = e_pad - e
    src_p = jnp.concatenate([src, jnp.zeros((pad,), jnp.int32)])
    dst_p = jnp.concatenate([dst, jnp.full((pad,), dummy_dst, jnp.int32)])
    return src_p.reshape(e_pad // _LN, _LN), dst_p.reshape(e_pad // _LN, _LN)


_RB = 1000
_EB = 512


def _dot(a, b):
    return jnp.dot(a, b, preferred_element_type=_F32)


def _linear_body(x_ref, w_ref, b_ref, o_ref):
    o_ref[...] = _dot(x_ref[...], w_ref[...]) + b_ref[...]


def _tc_linear(x, w, b):
    n, k = x.shape
    m = w.shape[1]
    return pl.pallas_call(
        _linear_body,
        grid=(pl.cdiv(n, _RB),),
        in_specs=[pl.BlockSpec((_RB, k), lambda i: (i, 0)),
                  pl.BlockSpec((k, m), lambda i: (0, 0)),
                  pl.BlockSpec((1, m), lambda i: (0, 0))],
        out_specs=pl.BlockSpec((_RB, m), lambda i: (i, 0)),
        out_shape=jax.ShapeDtypeStruct((n, m), _F32),
    )(x, w, b.reshape(1, -1))


def _sage_layer_body(h_ref, p_ref, c_ref, wl_ref, wr_ref, bl_ref, g_ref, b_ref,
                     o_ref):
    cnt = (jnp.sum(c_ref[0], axis=1, keepdims=True)
           + jnp.sum(c_ref[1], axis=1, keepdims=True))
    mean = (p_ref[0] + p_ref[1]) / jnp.maximum(cnt, 1.0)
    t = _dot(mean, wl_ref[...]) + bl_ref[...] + _dot(h_ref[...], wr_ref[...])
    t = jnp.maximum(t, 0.0)
    s = t + h_ref[...]
    mu = jnp.mean(s, axis=1, keepdims=True)
    var = jnp.mean((s - mu) ** 2, axis=1, keepdims=True)
    o_ref[...] = (s - mu) / jnp.sqrt(var + 1e-5) * g_ref[...] + b_ref[...]


def _tc_sage_layer(h, p, cnt, wl, bl, wr, g, b):
    n, d = h.shape
    n_acc = p.shape[0] // 2
    p3 = p.reshape(2, n_acc, d)
    c3 = cnt.reshape(2, n_acc, _LN)
    return pl.pallas_call(
        _sage_layer_body,
        grid=(n // _RB,),
        in_specs=[pl.BlockSpec((_RB, d), lambda i: (i, 0)),
                  pl.BlockSpec((2, _RB, d), lambda i: (0, i, 0)),
                  pl.BlockSpec((2, _RB, _LN), lambda i: (0, i, 0)),
                  pl.BlockSpec((d, d), lambda i: (0, 0)),
                  pl.BlockSpec((d, d), lambda i: (0, 0)),
                  pl.BlockSpec((1, d), lambda i: (0, 0)),
                  pl.BlockSpec((1, d), lambda i: (0, 0)),
                  pl.BlockSpec((1, d), lambda i: (0, 0))],
        out_specs=pl.BlockSpec((_RB, d), lambda i: (i, 0)),
        out_shape=jax.ShapeDtypeStruct((n, d), _F32),
    )(h, p3, c3, wl, wr, bl.reshape(1, -1), g.reshape(1, -1), b.reshape(1, -1))


def _mlp_body(x_ref, w1_ref, b1_ref, w2_ref, b2_ref, w3_ref, b3_ref, o_ref):
    h = jnp.maximum(_dot(x_ref[...], w1_ref[...]) + b1_ref[...], 0.0)
    h = jnp.maximum(_dot(h, w2_ref[...]) + b2_ref[...], 0.0)
    o_ref[...] = _dot(h, w3_ref[...]) + b3_ref[...]


def _tc_mlp(x, p):
    n, d = x.shape
    w3p = jnp.pad(p['W3'], ((0, 0), (0, 7)))
    b3p = jnp.pad(p['b3'].reshape(1, 1), ((0, 0), (0, 7)))
    return pl.pallas_call(
        _mlp_body,
        grid=(pl.cdiv(n, _RB),),
        in_specs=[pl.BlockSpec((_RB, d), lambda i: (i, 0)),
                  pl.BlockSpec((d, d), lambda i: (0, 0)),
                  pl.BlockSpec((1, d), lambda i: (0, 0)),
                  pl.BlockSpec((d, d), lambda i: (0, 0)),
                  pl.BlockSpec((1, d), lambda i: (0, 0)),
                  pl.BlockSpec((d, 8), lambda i: (0, 0)),
                  pl.BlockSpec((1, 8), lambda i: (0, 0))],
        out_specs=pl.BlockSpec((_RB, 8), lambda i: (i, 0)),
        out_shape=jax.ShapeDtypeStruct((n, 8), _F32),
    )(x, p['W1'], p['b1'].reshape(1, -1), p['W2'], p['b2'].reshape(1, -1),
      w3p, b3p)[:, 0]


def _edge_head_body(ga_ref, gb_ref, xe_ref, ew_ref, eb_ref, w1c_ref, b1_ref,
                    w2_ref, b2_ref, w3_ref, b3_ref, o_ref):
    he = jnp.maximum(_dot(xe_ref[...], ew_ref[...]) + eb_ref[...], 0.0)
    t = jnp.maximum(ga_ref[...] + gb_ref[...] + _dot(he, w1c_ref[...])
                    + b1_ref[...], 0.0)
    t = jnp.maximum(_dot(t, w2_ref[...]) + b2_ref[...], 0.0)
    o_ref[...] = _dot(t, w3_ref[...]) + b3_ref[...]


def _tc_edge_head(ga, gb, x_edge, ew, eb, w1c, hp):
    e, d = x_edge.shape[0], ga.shape[1]
    w3p = jnp.pad(hp['W3'], ((0, 0), (0, 7)))
    b3p = jnp.pad(hp['b3'].reshape(1, 1), ((0, 0), (0, 7)))
    return pl.pallas_call(
        _edge_head_body,
        grid=(pl.cdiv(e, _EB),),
        in_specs=[pl.BlockSpec((_EB, d), lambda i: (i, 0)),
                  pl.BlockSpec((_EB, d), lambda i: (i, 0)),
                  pl.BlockSpec((_EB, 16), lambda i: (i, 0)),
                  pl.BlockSpec((16, d), lambda i: (0, 0)),
                  pl.BlockSpec((1, d), lambda i: (0, 0)),
                  pl.BlockSpec((d, d), lambda i: (0, 0)),
                  pl.BlockSpec((1, d), lambda i: (0, 0)),
                  pl.BlockSpec((d, d), lambda i: (0, 0)),
                  pl.BlockSpec((1, d), lambda i: (0, 0)),
                  pl.BlockSpec((d, 8), lambda i: (0, 0)),
                  pl.BlockSpec((1, 8), lambda i: (0, 0))],
        out_specs=pl.BlockSpec((_EB, 8), lambda i: (i, 0)),
        out_shape=jax.ShapeDtypeStruct((e, 8), _F32),
    )(ga, gb, x_edge, ew, eb.reshape(1, -1), w1c,
      hp['b1'].reshape(1, -1), hp['W2'], hp['b2'].reshape(1, -1), w3p, b3p)[:, 0]


def _round_up(x, m):
    return (x + m - 1) // m * m


def _sage_stack(x, edge_index, p):
    n, d = x.shape
    e = edge_index.shape[1]
    n_acc = _round_up(n + 1, _NTILE * 64)
    e_pad = _round_up(e, _NW * 2 * _LN)
    src2, dst2 = _pad_edges(edge_index[0], edge_index[1], e_pad, n)
    cnt = _make_sc_cnt(e_pad, n_acc)(dst2)
    h = _tc_linear(x, p['in_W'], p['in_b'])
    for lp in p['layers']:
        part = _make_sc_agg(e_pad, n_acc, d)(h, src2, dst2)
        h = _tc_sage_layer(h, part, cnt, lp['Wl'], lp['bl'], lp['Wr'],
                           lp['g'], lp['b'])
    return h, src2, dst2, e_pad


def kernel(x2d, edge_index_2d, x1d, edge_index_1d, x_edge, params):
    p = params
    h2, _, _, _ = _sage_stack(x2d, edge_index_2d, p['gnn2d'])
    d2 = _tc_mlp(h2, p['head2d'])

    h1, src2, dst2, e_pad = _sage_stack(x1d, edge_index_1d, p['gnn1d'])
    d1 = _tc_mlp(h1, p['head1d'])
    inlet = _tc_mlp(h1, p['head_inlet'])

    w1 = p['head_edge']['W1']
    ab = _tc_linear(h1, jnp.concatenate([w1[:128], w1[128:256]], axis=1),
                    jnp.zeros((256,), _F32))
    a_rows = ab[:, :128]
    b_rows = ab[:, 128:]
    ga, gb = _make_sc_egather(e_pad, 128)(a_rows, b_rows, src2, dst2)
    eflow = _tc_edge_head(ga, gb, x_edge, p['edge_W'], p['edge_b'],
                          w1[256:384], p['head_edge'])
    return (d2, d1, inlet, eflow)
